# baseline (speedup 1.0000x reference)
import functools

import jax
import jax.numpy as jnp
from jax import lax
from jax.experimental import pallas as pl
from jax.experimental.pallas import tpu as pltpu
from jax.experimental.pallas import tpu_sc as plsc

_NC = 2
_NS = 16
_NW = _NC * _NS
_L = 16


def _sc_aggregate(x_pairs, rel_pairs, se, dst):
    _, pp, n = x_pairs.shape
    r = rel_pairs.shape[2]
    e = se.shape[0]
    dp = 2 * pp

    ch = 2000
    assert e % (2 * ch) == 0
    nch = e // ch
    gpc = ch // _L

    mesh = plsc.VectorSubcoreMesh(core_axis_name="c", subcore_axis_name="s")

    @functools.partial(
        pl.kernel,
        out_type=jax.ShapeDtypeStruct((_NW, dp, n), jnp.float32),
        mesh=mesh,
        compiler_params=pltpu.CompilerParams(needs_layout_passes=False),
        scratch_types=(
            [pltpu.VMEM((n,), jnp.int32) for _ in range(pp)]
            + [pltpu.VMEM((r,), jnp.int32) for _ in range(pp)]
            + [pltpu.VMEM((n,), jnp.float32) for _ in range(dp)]
            + [pltpu.VMEM((ch,), jnp.int32) for _ in range(4)]
            + [pltpu.SemaphoreType.DMA] * 3
        ),
    )
    def agg_kernel(x_hbm, rel_hbm, se_hbm, dst_hbm, out_hbm, *refs):
        xs = refs[0:pp]
        rs = refs[pp:2 * pp]
        accs = refs[2 * pp:2 * pp + dp]
        ebufs = refs[2 * pp + dp:2 * pp + dp + 4]
        sbufs = ebufs[0:2]
        dbufs = ebufs[2:4]
        sem_x, sem0, sem1 = refs[2 * pp + dp + 4:]
        sems = (sem0, sem1)

        cid = lax.axis_index("c")
        sid = lax.axis_index("s")
        wid = sid * _NC + cid

        for p in range(pp):
            pltpu.async_copy(x_hbm.at[wid, p], xs[p], sem_x)
            pltpu.async_copy(rel_hbm.at[wid, p], rs[p], sem_x)

        def zero_body(i, carry):
            for c in range(dp):
                accs[c][pl.ds(i * _L, _L)] = jnp.zeros((_L,), jnp.float32)
            return carry
        lax.fori_loop(0, n // _L, zero_body, 0, unroll=8)

        for p in range(pp):
            pltpu.make_async_copy(x_hbm.at[wid, p], xs[p], sem_x).wait()
            pltpu.make_async_copy(rel_hbm.at[wid, p], rs[p], sem_x).wait()

        def start(k, slot):
            off = k * ch
            pltpu.async_copy(se_hbm.at[pl.ds(off, ch)], sbufs[slot], sems[slot])
            pltpu.async_copy(dst_hbm.at[pl.ds(off, ch)], dbufs[slot], sems[slot])

        def wait(k, slot):
            off = k * ch
            pltpu.make_async_copy(se_hbm.at[pl.ds(off, ch)], sbufs[slot], sems[slot]).wait()
            pltpu.make_async_copy(dst_hbm.at[pl.ds(off, ch)], dbufs[slot], sems[slot]).wait()

        start(0, 0)

        himask = jnp.int32(-65536)

        def process(slot):
            sb, db = sbufs[slot], dbufs[slot]

            def compute(pk, d):
                s = lax.shift_right_logical(pk, 8)
                t = lax.bitwise_and(pk, jnp.int32(255))
                xw = [plsc.load_gather(xs[p], [s]) for p in range(pp)]
                rw = [plsc.load_gather(rs[p], [t]) for p in range(pp)]
                for p in range(pp):
                    xa = plsc.bitcast(lax.bitwise_and(xw[p], himask), jnp.float32)
                    xb = plsc.bitcast(lax.shift_left(xw[p], 16), jnp.float32)
                    ra = plsc.bitcast(lax.bitwise_and(rw[p], himask), jnp.float32)
                    rb = plsc.bitcast(lax.shift_left(rw[p], 16), jnp.float32)
                    plsc.addupdate_scatter(accs[2 * p], [d], xa - ra)
                    plsc.addupdate_scatter(accs[2 * p + 1], [d], xb - rb)

            def body(g, carry):
                pk, d = carry
                base = (g + 1) * _L
                pk2 = sb[pl.ds(base, _L)]
                d2 = db[pl.ds(base, _L)]
                compute(pk, d)
                return (pk2, d2)

            first = (sb[pl.ds(0, _L)], db[pl.ds(0, _L)])
            last = lax.fori_loop(0, gpc - 1, body, first, unroll=4)
            compute(*last)

        def outer(k2, carry):
            for b in range(2):
                kk = k2 * 2 + b

                @pl.when(kk + 1 < nch)
                def _():
                    start(kk + 1, 1 - b)

                wait(kk, b)
                process(b)
            return carry
        lax.fori_loop(0, nch // 2, outer, 0)

        for c in range(dp):
            pltpu.sync_copy(accs[c], out_hbm.at[wid, c])

    return agg_kernel(x_pairs, rel_pairs, se, dst)


def _pack_pairs(a):
    m, d = a.shape
    w = lax.bitcast_convert_type(a.T, jnp.int32)
    rnd = lax.bitwise_and(lax.shift_right_logical(w, 16), jnp.int32(1))
    bf = lax.shift_right_logical(w + rnd + jnp.int32(0x7FFF), 16)
    hi = lax.shift_left(bf[0::2], 16)
    lo = lax.bitwise_and(bf[1::2], jnp.int32(0xFFFF))
    return lax.bitwise_or(hi, lo).reshape(_NW, -1, m)


def _tc_finish(agg_n, w, norm):
    n, d = agg_n.shape
    nb = 2000
    assert n % nb == 0

    def body(a_ref, w_ref, nrm_ref, o_ref):
        o_ref[...] = jnp.dot(
            a_ref[...], w_ref[...], preferred_element_type=jnp.float32
        ) * nrm_ref[...]

    return pl.pallas_call(
        body,
        grid=(n // nb,),
        in_specs=[
            pl.BlockSpec((nb, d), lambda i: (i, 0)),
            pl.BlockSpec((d, d), lambda i: (0, 0)),
            pl.BlockSpec((nb, 1), lambda i: (i, 0)),
        ],
        out_specs=pl.BlockSpec((nb, d), lambda i: (i, 0)),
        out_shape=jax.ShapeDtypeStruct((n, d), jnp.float32),
    )(agg_n, w, norm)


def kernel(x, edge_index, edge_type, norm, emb_rel, pm_pd, W):
    n, d = x.shape

    x_pairs = _pack_pairs(x)
    rel_pairs = _pack_pairs(emb_rel)
    src = edge_index[0].astype(jnp.int32)
    dst = edge_index[1].astype(jnp.int32)
    et = edge_type.astype(jnp.int32)
    se = lax.shift_left(src, 8) | et

    agg = _sc_aggregate(x_pairs, rel_pairs, se, dst)
    agg_n = agg.reshape(d, n).T
    return _tc_finish(agg_n, W, norm)


# speedup vs baseline: 1.5683x; 1.5683x over previous
import functools

import jax
import jax.numpy as jnp
from jax import lax
from jax.experimental import pallas as pl
from jax.experimental.pallas import tpu as pltpu
from jax.experimental.pallas import tpu_sc as plsc

_NC = 2
_NS = 16
_NW = _NC * _NS
_L = 16


def _sc_aggregate(x_pairs, rel_pairs, se, dst):
    _, pp, n = x_pairs.shape
    r = rel_pairs.shape[2]
    e = se.shape[0]
    dp = 2 * pp

    ch = 2000
    assert e % (2 * ch) == 0
    nch = e // ch
    gpc = ch // _L

    mesh = plsc.VectorSubcoreMesh(core_axis_name="c", subcore_axis_name="s")

    @functools.partial(
        pl.kernel,
        out_type=jax.ShapeDtypeStruct((_NW, dp, n), jnp.float32),
        mesh=mesh,
        compiler_params=pltpu.CompilerParams(needs_layout_passes=False),
        scratch_types=(
            [pltpu.VMEM((n,), jnp.int32) for _ in range(pp)]
            + [pltpu.VMEM((r,), jnp.int32) for _ in range(pp)]
            + [pltpu.VMEM((n,), jnp.float32) for _ in range(dp)]
            + [pltpu.VMEM((ch,), jnp.int32) for _ in range(4)]
            + [pltpu.SemaphoreType.DMA] * 3
        ),
    )
    def agg_kernel(x_hbm, rel_hbm, se_hbm, dst_hbm, out_hbm, *refs):
        xs = refs[0:pp]
        rs = refs[pp:2 * pp]
        accs = refs[2 * pp:2 * pp + dp]
        ebufs = refs[2 * pp + dp:2 * pp + dp + 4]
        sbufs = ebufs[0:2]
        dbufs = ebufs[2:4]
        sem_x, sem0, sem1 = refs[2 * pp + dp + 4:]
        sems = (sem0, sem1)

        cid = lax.axis_index("c")
        sid = lax.axis_index("s")
        wid = sid * _NC + cid

        for p in range(pp):
            pltpu.async_copy(x_hbm.at[wid, p], xs[p], sem_x)
            pltpu.async_copy(rel_hbm.at[wid, p], rs[p], sem_x)

        def zero_body(i, carry):
            for c in range(dp):
                accs[c][pl.ds(i * _L, _L)] = jnp.zeros((_L,), jnp.float32)
            return carry
        lax.fori_loop(0, n // _L, zero_body, 0, unroll=8)

        for p in range(pp):
            pltpu.make_async_copy(x_hbm.at[wid, p], xs[p], sem_x).wait()
            pltpu.make_async_copy(rel_hbm.at[wid, p], rs[p], sem_x).wait()

        def start(k, slot):
            off = k * ch
            pltpu.async_copy(se_hbm.at[pl.ds(off, ch)], sbufs[slot], sems[slot])
            pltpu.async_copy(dst_hbm.at[pl.ds(off, ch)], dbufs[slot], sems[slot])

        def wait(k, slot):
            off = k * ch
            pltpu.make_async_copy(se_hbm.at[pl.ds(off, ch)], sbufs[slot], sems[slot]).wait()
            pltpu.make_async_copy(dst_hbm.at[pl.ds(off, ch)], dbufs[slot], sems[slot]).wait()

        start(0, 0)

        himask = jnp.int32(-65536)

        def process(slot):
            sb, db = sbufs[slot], dbufs[slot]

            def compute(pk, d):
                s = lax.shift_right_logical(pk, 8)
                t = lax.bitwise_and(pk, jnp.int32(255))
                xw = [plsc.load_gather(xs[p], [s]) for p in range(pp)]
                rw = [plsc.load_gather(rs[p], [t]) for p in range(pp)]
                for p in range(pp):
                    xa = plsc.bitcast(lax.bitwise_and(xw[p], himask), jnp.float32)
                    xb = plsc.bitcast(lax.shift_left(xw[p], 16), jnp.float32)
                    ra = plsc.bitcast(lax.bitwise_and(rw[p], himask), jnp.float32)
                    rb = plsc.bitcast(lax.shift_left(rw[p], 16), jnp.float32)
                    plsc.addupdate_scatter(accs[2 * p], [d], xa - ra)
                    plsc.addupdate_scatter(accs[2 * p + 1], [d], xb - rb)

            def body(g, carry):
                pk, d = carry
                base = (g + 1) * _L
                pk2 = sb[pl.ds(base, _L)]
                d2 = db[pl.ds(base, _L)]
                compute(pk, d)
                return (pk2, d2)

            first = (sb[pl.ds(0, _L)], db[pl.ds(0, _L)])
            last = lax.fori_loop(0, gpc - 1, body, first, unroll=4)
            compute(*last)

        def outer(k2, carry):
            for b in range(2):
                kk = k2 * 2 + b

                @pl.when(kk + 1 < nch)
                def _():
                    start(kk + 1, 1 - b)

                wait(kk, b)
                process(b)
            return carry
        lax.fori_loop(0, nch // 2, outer, 0)

        for c in range(dp):
            pltpu.sync_copy(accs[c], out_hbm.at[wid, c])

    return agg_kernel(x_pairs, rel_pairs, se, dst)


def _pack_pairs(a):
    m, d = a.shape
    w = lax.bitcast_convert_type(a.T, jnp.int32)
    rnd = lax.bitwise_and(lax.shift_right_logical(w, 16), jnp.int32(1))
    bf = lax.shift_right_logical(w + rnd + jnp.int32(0x7FFF), 16)
    bf2 = lax.optimization_barrier(bf).reshape(d // 2, 2 * m)
    hi = lax.shift_left(bf2[:, :m], 16)
    lo = lax.bitwise_and(bf2[:, m:], jnp.int32(0xFFFF))
    return lax.bitwise_or(hi, lo).reshape(_NW, -1, m)


def _tc_finish(agg_n, w, norm):
    n, d = agg_n.shape
    nb = 2000
    assert n % nb == 0

    def body(a_ref, w_ref, nrm_ref, o_ref):
        o_ref[...] = jnp.dot(
            a_ref[...], w_ref[...], preferred_element_type=jnp.float32
        ) * nrm_ref[...]

    return pl.pallas_call(
        body,
        grid=(n // nb,),
        in_specs=[
            pl.BlockSpec((nb, d), lambda i: (i, 0)),
            pl.BlockSpec((d, d), lambda i: (0, 0)),
            pl.BlockSpec((nb, 1), lambda i: (i, 0)),
        ],
        out_specs=pl.BlockSpec((nb, d), lambda i: (i, 0)),
        out_shape=jax.ShapeDtypeStruct((n, d), jnp.float32),
    )(agg_n, w, norm)


def kernel(x, edge_index, edge_type, norm, emb_rel, pm_pd, W):
    n, d = x.shape

    x_pairs = _pack_pairs(x)
    rel_pairs = _pack_pairs(emb_rel)
    src = edge_index[0].astype(jnp.int32)
    dst = edge_index[1].astype(jnp.int32)
    et = edge_type.astype(jnp.int32)
    se = lax.shift_left(src, 8) | et

    agg = _sc_aggregate(x_pairs, rel_pairs, se, dst)
    agg_n = agg.reshape(d, n).T
    return _tc_finish(agg_n, W, norm)


# speedup vs baseline: 1.5780x; 1.0062x over previous
import functools

import jax
import jax.numpy as jnp
from jax import lax
from jax.experimental import pallas as pl
from jax.experimental.pallas import tpu as pltpu
from jax.experimental.pallas import tpu_sc as plsc

_NC = 2
_NS = 16
_NW = _NC * _NS
_L = 16


def _sc_aggregate(x_pairs, rel_pairs, se, dst):
    _, pp, n = x_pairs.shape
    r = rel_pairs.shape[2]
    e = se.shape[0]
    dp = 2 * pp

    ch = 4000
    assert e % (2 * ch) == 0
    nch = e // ch
    gpc = ch // _L

    mesh = plsc.VectorSubcoreMesh(core_axis_name="c", subcore_axis_name="s")

    @functools.partial(
        pl.kernel,
        out_type=jax.ShapeDtypeStruct((_NW, dp, n), jnp.float32),
        mesh=mesh,
        compiler_params=pltpu.CompilerParams(needs_layout_passes=False),
        scratch_types=(
            [pltpu.VMEM((n,), jnp.int32) for _ in range(pp)]
            + [pltpu.VMEM((r,), jnp.int32) for _ in range(pp)]
            + [pltpu.VMEM((n,), jnp.float32) for _ in range(dp)]
            + [pltpu.VMEM((ch,), jnp.int32) for _ in range(4)]
            + [pltpu.SemaphoreType.DMA] * 3
        ),
    )
    def agg_kernel(x_hbm, rel_hbm, se_hbm, dst_hbm, out_hbm, *refs):
        xs = refs[0:pp]
        rs = refs[pp:2 * pp]
        accs = refs[2 * pp:2 * pp + dp]
        ebufs = refs[2 * pp + dp:2 * pp + dp + 4]
        sbufs = ebufs[0:2]
        dbufs = ebufs[2:4]
        sem_x, sem0, sem1 = refs[2 * pp + dp + 4:]
        sems = (sem0, sem1)

        cid = lax.axis_index("c")
        sid = lax.axis_index("s")
        wid = sid * _NC + cid

        for p in range(pp):
            pltpu.async_copy(x_hbm.at[wid, p], xs[p], sem_x)
            pltpu.async_copy(rel_hbm.at[wid, p], rs[p], sem_x)

        def zero_body(i, carry):
            for c in range(dp):
                accs[c][pl.ds(i * _L, _L)] = jnp.zeros((_L,), jnp.float32)
            return carry
        lax.fori_loop(0, n // _L, zero_body, 0, unroll=8)

        for p in range(pp):
            pltpu.make_async_copy(x_hbm.at[wid, p], xs[p], sem_x).wait()
            pltpu.make_async_copy(rel_hbm.at[wid, p], rs[p], sem_x).wait()

        def start(k, slot):
            off = k * ch
            pltpu.async_copy(se_hbm.at[pl.ds(off, ch)], sbufs[slot], sems[slot])
            pltpu.async_copy(dst_hbm.at[pl.ds(off, ch)], dbufs[slot], sems[slot])

        def wait(k, slot):
            off = k * ch
            pltpu.make_async_copy(se_hbm.at[pl.ds(off, ch)], sbufs[slot], sems[slot]).wait()
            pltpu.make_async_copy(dst_hbm.at[pl.ds(off, ch)], dbufs[slot], sems[slot]).wait()

        start(0, 0)

        himask = jnp.int32(-65536)

        def process(slot):
            sb, db = sbufs[slot], dbufs[slot]

            def compute(pk, d):
                s = lax.shift_right_logical(pk, 8)
                t = lax.bitwise_and(pk, jnp.int32(255))
                xw = [plsc.load_gather(xs[p], [s]) for p in range(pp)]
                rw = [plsc.load_gather(rs[p], [t]) for p in range(pp)]
                for p in range(pp):
                    xa = plsc.bitcast(lax.bitwise_and(xw[p], himask), jnp.float32)
                    xb = plsc.bitcast(lax.shift_left(xw[p], 16), jnp.float32)
                    ra = plsc.bitcast(lax.bitwise_and(rw[p], himask), jnp.float32)
                    rb = plsc.bitcast(lax.shift_left(rw[p], 16), jnp.float32)
                    plsc.addupdate_scatter(accs[2 * p], [d], xa - ra)
                    plsc.addupdate_scatter(accs[2 * p + 1], [d], xb - rb)

            def body(g, carry):
                pk, d = carry
                base = (g + 1) * _L
                pk2 = sb[pl.ds(base, _L)]
                d2 = db[pl.ds(base, _L)]
                compute(pk, d)
                return (pk2, d2)

            first = (sb[pl.ds(0, _L)], db[pl.ds(0, _L)])
            last = lax.fori_loop(0, gpc - 1, body, first, unroll=8)
            compute(*last)

        def outer(k2, carry):
            for b in range(2):
                kk = k2 * 2 + b

                @pl.when(kk + 1 < nch)
                def _():
                    start(kk + 1, 1 - b)

                wait(kk, b)
                process(b)
            return carry
        lax.fori_loop(0, nch // 2, outer, 0)

        for c in range(dp):
            pltpu.sync_copy(accs[c], out_hbm.at[wid, c])

    return agg_kernel(x_pairs, rel_pairs, se, dst)


def _pack_pairs(a):
    m, d = a.shape
    w = lax.bitcast_convert_type(a.T, jnp.int32)
    rnd = lax.bitwise_and(lax.shift_right_logical(w, 16), jnp.int32(1))
    bf = lax.shift_right_logical(w + rnd + jnp.int32(0x7FFF), 16)
    bf2 = lax.optimization_barrier(bf).reshape(d // 2, 2 * m)
    hi = lax.shift_left(bf2[:, :m], 16)
    lo = lax.bitwise_and(bf2[:, m:], jnp.int32(0xFFFF))
    return lax.bitwise_or(hi, lo).reshape(_NW, -1, m)


def _tc_finish(agg_n, w, norm):
    n, d = agg_n.shape
    nb = 2000
    assert n % nb == 0

    def body(a_ref, w_ref, nrm_ref, o_ref):
        o_ref[...] = jnp.dot(
            a_ref[...], w_ref[...], preferred_element_type=jnp.float32
        ) * nrm_ref[...]

    return pl.pallas_call(
        body,
        grid=(n // nb,),
        in_specs=[
            pl.BlockSpec((nb, d), lambda i: (i, 0)),
            pl.BlockSpec((d, d), lambda i: (0, 0)),
            pl.BlockSpec((nb, 1), lambda i: (i, 0)),
        ],
        out_specs=pl.BlockSpec((nb, d), lambda i: (i, 0)),
        out_shape=jax.ShapeDtypeStruct((n, d), jnp.float32),
    )(agg_n, w, norm)


def kernel(x, edge_index, edge_type, norm, emb_rel, pm_pd, W):
    n, d = x.shape

    x_pairs = _pack_pairs(x)
    rel_pairs = _pack_pairs(emb_rel)
    src = edge_index[0].astype(jnp.int32)
    dst = edge_index[1].astype(jnp.int32)
    et = edge_type.astype(jnp.int32)
    se = lax.shift_left(src, 8) | et

    agg = _sc_aggregate(x_pairs, rel_pairs, se, dst)
    agg_n = agg.reshape(d, n).T
    return _tc_finish(agg_n, W, norm)


# speedup vs baseline: 1.9486x; 1.2349x over previous
import functools

import jax
import jax.numpy as jnp
from jax import lax
from jax.experimental import pallas as pl
from jax.experimental.pallas import tpu as pltpu
from jax.experimental.pallas import tpu_sc as plsc

_NC = 2
_NS = 16
_NW = _NC * _NS
_L = 16


def _sc_aggregate(x_pairs, rel_pairs, se, dst):
    _, pp, n = x_pairs.shape
    r = rel_pairs.shape[2]
    e = se.shape[0]
    dp = 2 * pp

    ch = 4000
    assert e % (2 * ch) == 0
    nch = e // ch
    gpc = ch // _L

    mesh = plsc.VectorSubcoreMesh(core_axis_name="c", subcore_axis_name="s")

    @functools.partial(
        pl.kernel,
        out_type=jax.ShapeDtypeStruct((_NW, dp, n), jnp.float32),
        mesh=mesh,
        compiler_params=pltpu.CompilerParams(needs_layout_passes=False),
        scratch_types=(
            [pltpu.VMEM((n,), jnp.int32) for _ in range(pp)]
            + [pltpu.VMEM((r,), jnp.int32) for _ in range(pp)]
            + [pltpu.VMEM((n,), jnp.float32) for _ in range(dp)]
            + [pltpu.VMEM((ch,), jnp.int32) for _ in range(4)]
            + [pltpu.SemaphoreType.DMA] * 3
        ),
    )
    def agg_kernel(x_hbm, rel_hbm, se_hbm, dst_hbm, out_hbm, *refs):
        xs = refs[0:pp]
        rs = refs[pp:2 * pp]
        accs = refs[2 * pp:2 * pp + dp]
        ebufs = refs[2 * pp + dp:2 * pp + dp + 4]
        sbufs = ebufs[0:2]
        dbufs = ebufs[2:4]
        sem_x, sem0, sem1 = refs[2 * pp + dp + 4:]
        sems = (sem0, sem1)

        cid = lax.axis_index("c")
        sid = lax.axis_index("s")
        wid = sid * _NC + cid

        for p in range(pp):
            pltpu.async_copy(x_hbm.at[wid, p], xs[p], sem_x)
            pltpu.async_copy(rel_hbm.at[wid, p], rs[p], sem_x)

        def zero_body(i, carry):
            for c in range(dp):
                accs[c][pl.ds(i * _L, _L)] = jnp.zeros((_L,), jnp.float32)
            return carry
        lax.fori_loop(0, n // _L, zero_body, 0, unroll=8)

        for p in range(pp):
            pltpu.make_async_copy(x_hbm.at[wid, p], xs[p], sem_x).wait()
            pltpu.make_async_copy(rel_hbm.at[wid, p], rs[p], sem_x).wait()

        def start(k, slot):
            off = k * ch
            pltpu.async_copy(se_hbm.at[pl.ds(off, ch)], sbufs[slot], sems[slot])
            pltpu.async_copy(dst_hbm.at[pl.ds(off, ch)], dbufs[slot], sems[slot])

        def wait(k, slot):
            off = k * ch
            pltpu.make_async_copy(se_hbm.at[pl.ds(off, ch)], sbufs[slot], sems[slot]).wait()
            pltpu.make_async_copy(dst_hbm.at[pl.ds(off, ch)], dbufs[slot], sems[slot]).wait()

        start(0, 0)

        himask = jnp.int32(-65536)

        def process(slot):
            sb, db = sbufs[slot], dbufs[slot]

            def load_idx(g):
                base = g * _L
                return sb[pl.ds(base, _L)], db[pl.ds(base, _L)]

            def gather(pk):
                s = lax.shift_right_logical(pk, 8)
                t = lax.bitwise_and(pk, jnp.int32(255))
                xw = [plsc.load_gather(xs[p], [s]) for p in range(pp)]
                rw = [plsc.load_gather(rs[p], [t]) for p in range(pp)]
                return xw, rw

            def compute(xw, rw, d):
                for p in range(pp):
                    xa = plsc.bitcast(lax.shift_left(xw[p], 16), jnp.float32)
                    xb = plsc.bitcast(lax.bitwise_and(xw[p], himask), jnp.float32)
                    ra = plsc.bitcast(lax.shift_left(rw[p], 16), jnp.float32)
                    rb = plsc.bitcast(lax.bitwise_and(rw[p], himask), jnp.float32)
                    plsc.addupdate_scatter(accs[2 * p], [d], xa - ra)
                    plsc.addupdate_scatter(accs[2 * p + 1], [d], xb - rb)

            def body(g, carry):
                pk1, d1, xw, rw, d0 = carry
                pk2, d2 = load_idx(g + 2)
                xw1, rw1 = gather(pk1)
                compute(xw, rw, d0)
                return (pk2, d2, xw1, rw1, d1)

            pk0, d0 = load_idx(0)
            pk1, d1 = load_idx(1)
            xw0, rw0 = gather(pk0)
            carry = (pk1, d1, xw0, rw0, d0)
            carry = lax.fori_loop(0, gpc - 2, body, carry, unroll=8)
            pk1, d1, xw, rw, d0 = carry
            xw1, rw1 = gather(pk1)
            compute(xw, rw, d0)
            compute(xw1, rw1, d1)

        def outer(k2, carry):
            for b in range(2):
                kk = k2 * 2 + b

                @pl.when(kk + 1 < nch)
                def _():
                    start(kk + 1, 1 - b)

                wait(kk, b)
                process(b)
            return carry
        lax.fori_loop(0, nch // 2, outer, 0)

        for c in range(dp):
            pltpu.sync_copy(accs[c], out_hbm.at[wid, c])

    return agg_kernel(x_pairs, rel_pairs, se, dst)


def _pack_pairs(a):
    m, d = a.shape
    bf = a.astype(jnp.bfloat16).reshape(m, d // 2, 2)
    words = lax.bitcast_convert_type(bf, jnp.int32)
    return words.T.reshape(_NW, -1, m)


def _tc_finish(agg_n, w, norm):
    n, d = agg_n.shape
    nb = 2000
    assert n % nb == 0

    def body(a_ref, w_ref, nrm_ref, o_ref):
        o_ref[...] = jnp.dot(
            a_ref[...], w_ref[...], preferred_element_type=jnp.float32
        ) * nrm_ref[...]

    return pl.pallas_call(
        body,
        grid=(n // nb,),
        in_specs=[
            pl.BlockSpec((nb, d), lambda i: (i, 0)),
            pl.BlockSpec((d, d), lambda i: (0, 0)),
            pl.BlockSpec((nb, 1), lambda i: (i, 0)),
        ],
        out_specs=pl.BlockSpec((nb, d), lambda i: (i, 0)),
        out_shape=jax.ShapeDtypeStruct((n, d), jnp.float32),
    )(agg_n, w, norm)


def kernel(x, edge_index, edge_type, norm, emb_rel, pm_pd, W):
    n, d = x.shape

    x_pairs = _pack_pairs(x)
    rel_pairs = _pack_pairs(emb_rel)
    src = edge_index[0].astype(jnp.int32)
    dst = edge_index[1].astype(jnp.int32)
    et = edge_type.astype(jnp.int32)
    se = lax.shift_left(src, 8) | et

    agg = _sc_aggregate(x_pairs, rel_pairs, se, dst)
    agg_n = agg.reshape(d, n).T
    return _tc_finish(agg_n, W, norm)
